# baseline (device time: 35610 ns/iter reference)
import jax
import jax.numpy as jnp
from jax import lax
from jax.experimental import pallas as pl
from jax.experimental.pallas import tpu as pltpu

N_DEV = 8

MASK_X, MASK_Y, MASK_Z = 1, 3, 4

ORDER_XYZ = (MASK_X, MASK_Y, MASK_Z)
ORDER_YZX = (MASK_Y, MASK_Z, MASK_X)
ORDER_ZXY = (MASK_Z, MASK_X, MASK_Y)

CHUNK = 128
SUB = CHUNK // 8
CHUNKS = (
    (0, ORDER_XYZ),
    (128, ORDER_YZX),
    (256, ORDER_ZXY),
    (384, ORDER_XYZ),
    (512, ORDER_YZX),
    (640, ORDER_ZXY),
    (768, ORDER_XYZ),
    (896, ORDER_YZX),
)
N_C = len(CHUNKS)

SLOTS = 11

BC_COEFFS = (
    (1, 0, 0), (0, 1, 0), (0, 0, 1),
    (1, 1, 0), (1, 0, 1), (0, 1, 1), (1, 1, 1),
)


def kernel(A, B):
    m, k = A.shape
    _, n = B.shape

    def body(a_ref, b_ref, out_hbm, work_ref, fout_ref, a16_ref, b16_ref, *scratch):
        r0_bufs = scratch[:N_C]
        r1_bufs = scratch[N_C : 4 * N_C]
        send_sems = scratch[4 * N_C]
        recv_sems = scratch[4 * N_C + 1]
        copy_sems = scratch[4 * N_C + 2]

        my = lax.axis_index("i")
        vx = (my ^ (my >> 1)) & 1
        vy = (my >> 1) & 1
        vz = (my >> 2) & 1
        bit_of = {MASK_X: vx, MASK_Y: vy, MASK_Z: vz}

        peers = tuple(range(1, N_DEV))
        barrier_sem = pltpu.get_barrier_semaphore()
        for mask in peers:
            pl.semaphore_signal(
                barrier_sem, inc=1,
                device_id=(my ^ mask,),
                device_id_type=pl.DeviceIdType.MESH,
            )

        all_rdmas = []
        copy_ops = []

        def start_rdma(src, dst, slot, partner):
            rdma = pltpu.make_async_remote_copy(
                src_ref=src,
                dst_ref=dst,
                send_sem=send_sems.at[slot],
                recv_sem=recv_sems.at[slot],
                device_id=(partner,),
                device_id_type=pl.DeviceIdType.MESH,
            )
            rdma.start()
            all_rdmas.append(rdma)
            return rdma

        def emit_block(c, slot, lo, ln, values=None):
            base, _ = CHUNKS[c]
            if values is None:
                values = work_ref[pl.ds(base + lo, ln), :]
            fout_ref[pl.ds(base + lo, ln), :] = values.astype(jnp.float32)
            cp = pltpu.make_async_copy(
                fout_ref.at[pl.ds(base + lo, ln), :],
                out_hbm.at[pl.ds(base + lo, ln), :],
                copy_sems.at[c * 8 + slot],
            )
            cp.start()
            copy_ops.append(cp)

        b16_ref[:, :] = b_ref[:, :].astype(jnp.bfloat16)
        a16_ref[:, :] = a_ref[:, :].astype(jnp.bfloat16)
        rdmas = [None] * N_C
        keep = [None] * N_C
        subs = [None] * N_C
        for c in range(N_C):
            base, masks = CHUNKS[c]
            work_ref[base : base + CHUNK, :] = jnp.dot(
                a16_ref[base : base + CHUNK, :],
                b16_ref[:, :],
                preferred_element_type=jnp.float32,
            ).astype(jnp.bfloat16)
            if c == 0:
                pl.semaphore_wait(barrier_sem, len(peers))
            b1 = bit_of[masks[0]]
            half = CHUNK // 2
            send_lo = (1 - b1) * half
            keep[c] = b1 * half
            rdmas[c] = start_rdma(
                work_ref.at[pl.ds(base + send_lo, half), :],
                r0_bufs[c],
                c * SLOTS + 0,
                my ^ masks[0],
            )

        for c in range(N_C):
            base, masks = CHUNKS[c]
            half = CHUNK // 2
            rdmas[c].wait_recv()
            work_ref[pl.ds(base + keep[c], half), :] += r0_bufs[c][:, :]
            b2, b3 = bit_of[masks[1]], bit_of[masks[2]]
            s_me = 2 * b2 + b3
            s2 = 2 * (1 - b2) + b3
            s3 = 2 * b2 + (1 - b3)
            s23 = 2 * (1 - b2) + (1 - b3)
            subs[c] = (s_me, s2, s3, s23)
            partners = (my ^ masks[1], my ^ masks[2], my ^ masks[1] ^ masks[2])
            group = []
            for j, (s, p) in enumerate(zip((s2, s3, s23), partners)):
                group.append(start_rdma(
                    work_ref.at[pl.ds(base + keep[c] + s * SUB, SUB), :],
                    r1_bufs[c * 3 + j],
                    c * SLOTS + 1 + j,
                    p,
                ))
            rdmas[c] = group

        for c in range(N_C):
            base, masks = CHUNKS[c]
            s_me, _, _, _ = subs[c]
            for rdma in rdmas[c]:
                rdma.wait_recv()
            my_lo = keep[c] + s_me * SUB
            blk = jnp.maximum(
                work_ref[pl.ds(base + my_lo, SUB), :]
                + r1_bufs[c * 3 + 0][:, :]
                + r1_bufs[c * 3 + 1][:, :]
                + r1_bufs[c * 3 + 2][:, :],
                jnp.bfloat16(0.0),
            )
            work_ref[pl.ds(base + my_lo, SUB), :] = blk
            group = []
            for j, (a1, a2, a3) in enumerate(BC_COEFFS):
                mu = a1 * masks[0] ^ a2 * masks[1] ^ a3 * masks[2]
                group.append(start_rdma(
                    work_ref.at[pl.ds(base + my_lo, SUB), :],
                    work_ref.at[pl.ds(base + my_lo, SUB), :],
                    c * SLOTS + 4 + j,
                    my ^ mu,
                ))
            rdmas[c] = group
            emit_block(c, 0, my_lo, SUB, values=blk)

        for c in range(N_C):
            base, masks = CHUNKS[c]
            b1 = bit_of[masks[0]]
            b2 = bit_of[masks[1]]
            b3 = bit_of[masks[2]]
            for j, (a1, a2, a3) in enumerate(BC_COEFFS):
                rdmas[c][j].wait_recv()
                lo_q = (b1 ^ a1) * (CHUNK // 2) + (2 * (b2 ^ a2) + (b3 ^ a3)) * SUB
                emit_block(c, 1 + j, lo_q, SUB)

        for rdma in all_rdmas:
            rdma.wait_send()
        for cp in copy_ops:
            cp.wait()

    scratch_shapes = [
        pltpu.VMEM((m, n), jnp.bfloat16),
        pltpu.VMEM((m, n), jnp.float32),
        pltpu.VMEM((m, k), jnp.bfloat16),
        pltpu.VMEM((k, n), jnp.bfloat16),
    ] + [
        pltpu.VMEM((CHUNK // 2, n), jnp.bfloat16) for _ in range(N_C)
    ] + [
        pltpu.VMEM((SUB, n), jnp.bfloat16) for _ in range(3 * N_C)
    ] + [
        pltpu.SemaphoreType.DMA((SLOTS * N_C,)),
        pltpu.SemaphoreType.DMA((SLOTS * N_C,)),
        pltpu.SemaphoreType.DMA((8 * N_C,)),
    ]

    return pl.pallas_call(
        body,
        out_shape=jax.ShapeDtypeStruct((m, n), jnp.float32),
        in_specs=[
            pl.BlockSpec(memory_space=pltpu.VMEM),
            pl.BlockSpec(memory_space=pltpu.VMEM),
        ],
        out_specs=pl.BlockSpec(memory_space=pl.ANY),
        scratch_shapes=scratch_shapes,
        compiler_params=pltpu.CompilerParams(
            vmem_limit_bytes=100 * 1024 * 1024,
            collective_id=0,
        ),
    )(A, B)


# device time: 31084 ns/iter; 1.1456x vs baseline; 1.1456x over previous
import jax
import jax.numpy as jnp
from jax import lax
from jax.experimental import pallas as pl
from jax.experimental.pallas import tpu as pltpu

N_DEV = 8

MASK_X, MASK_Y, MASK_Z = 1, 3, 4

ORDER_XYZ = (MASK_X, MASK_Y, MASK_Z)
ORDER_YZX = (MASK_Y, MASK_Z, MASK_X)
ORDER_ZXY = (MASK_Z, MASK_X, MASK_Y)

CHUNK = 128
SUB = CHUNK // 8
CHUNKS = (
    (0, ORDER_XYZ),
    (128, ORDER_YZX),
    (256, ORDER_ZXY),
    (384, ORDER_XYZ),
    (512, ORDER_YZX),
    (640, ORDER_ZXY),
    (768, ORDER_XYZ),
    (896, ORDER_YZX),
)
N_C = len(CHUNKS)

SLOTS = 8


def kernel(A, B):
    m, k = A.shape
    _, n = B.shape

    def body(a_ref, b_ref, out_hbm, work_ref, fout_ref, *scratch):
        r0_bufs = scratch[:N_C]
        r1_bufs = scratch[N_C : 4 * N_C]
        send_sems = scratch[4 * N_C]
        recv_sems = scratch[4 * N_C + 1]
        copy_sems = scratch[4 * N_C + 2]

        my = lax.axis_index("i")
        vx = (my ^ (my >> 1)) & 1
        vy = (my >> 1) & 1
        vz = (my >> 2) & 1
        bit_of = {MASK_X: vx, MASK_Y: vy, MASK_Z: vz}

        peers = (MASK_X, MASK_Y, MASK_Z, MASK_X ^ MASK_Y, MASK_Y ^ MASK_Z,
                 MASK_Z ^ MASK_X)
        barrier_sem = pltpu.get_barrier_semaphore()
        for mask in peers:
            pl.semaphore_signal(
                barrier_sem, inc=1,
                device_id=(my ^ mask,),
                device_id_type=pl.DeviceIdType.MESH,
            )

        all_rdmas = []
        copy_ops = []

        def start_rdma(src, dst, slot, partner):
            rdma = pltpu.make_async_remote_copy(
                src_ref=src,
                dst_ref=dst,
                send_sem=send_sems.at[slot],
                recv_sem=recv_sems.at[slot],
                device_id=(partner,),
                device_id_type=pl.DeviceIdType.MESH,
            )
            rdma.start()
            all_rdmas.append(rdma)
            return rdma

        def emit_block(c, slot, lo, ln, values=None):
            base, _ = CHUNKS[c]
            if values is None:
                values = work_ref[pl.ds(base + lo, ln), :]
            fout_ref[pl.ds(base + lo, ln), :] = values.astype(jnp.float32)
            cp = pltpu.make_async_copy(
                fout_ref.at[pl.ds(base + lo, ln), :],
                out_hbm.at[pl.ds(base + lo, ln), :],
                copy_sems.at[c * 5 + slot],
            )
            cp.start()
            copy_ops.append(cp)

        rdmas = [None] * N_C
        keep = [None] * N_C
        subs = [None] * N_C
        for c in range(N_C):
            base, masks = CHUNKS[c]
            work_ref[base : base + CHUNK, :] = jnp.dot(
                a_ref[base : base + CHUNK, :],
                b_ref[:, :],
                preferred_element_type=jnp.float32,
            ).astype(jnp.bfloat16)
            if c == 0:
                pl.semaphore_wait(barrier_sem, len(peers))
            b1 = bit_of[masks[0]]
            half = CHUNK // 2
            send_lo = (1 - b1) * half
            keep[c] = b1 * half
            rdmas[c] = start_rdma(
                work_ref.at[pl.ds(base + send_lo, half), :],
                r0_bufs[c],
                c * SLOTS + 0,
                my ^ masks[0],
            )

        for c in range(N_C):
            base, masks = CHUNKS[c]
            half = CHUNK // 2
            rdmas[c].wait_recv()
            work_ref[pl.ds(base + keep[c], half), :] += r0_bufs[c][:, :]
            b2, b3 = bit_of[masks[1]], bit_of[masks[2]]
            s_me = 2 * b2 + b3
            s2 = 2 * (1 - b2) + b3
            s3 = 2 * b2 + (1 - b3)
            s23 = 2 * (1 - b2) + (1 - b3)
            subs[c] = (s_me, s2, s3, s23)
            partners = (my ^ masks[1], my ^ masks[2], my ^ masks[1] ^ masks[2])
            group = []
            for j, (s, p) in enumerate(zip((s2, s3, s23), partners)):
                group.append(start_rdma(
                    work_ref.at[pl.ds(base + keep[c] + s * SUB, SUB), :],
                    r1_bufs[c * 3 + j],
                    c * SLOTS + 1 + j,
                    p,
                ))
            rdmas[c] = group

        for c in range(N_C):
            base, masks = CHUNKS[c]
            s_me, s2, s3, s23 = subs[c]
            for rdma in rdmas[c]:
                rdma.wait_recv()
            my_lo = keep[c] + s_me * SUB
            blk = jnp.maximum(
                work_ref[pl.ds(base + my_lo, SUB), :]
                + r1_bufs[c * 3 + 0][:, :]
                + r1_bufs[c * 3 + 1][:, :]
                + r1_bufs[c * 3 + 2][:, :],
                jnp.bfloat16(0.0),
            )
            work_ref[pl.ds(base + my_lo, SUB), :] = blk
            partners = (my ^ masks[1], my ^ masks[2], my ^ masks[1] ^ masks[2])
            group = []
            for j, p in enumerate(partners):
                group.append(start_rdma(
                    work_ref.at[pl.ds(base + my_lo, SUB), :],
                    work_ref.at[pl.ds(base + my_lo, SUB), :],
                    c * SLOTS + 4 + j,
                    p,
                ))
            rdmas[c] = group
            emit_block(c, 0, my_lo, SUB, values=blk)

        for c in range(N_C):
            base, masks = CHUNKS[c]
            s_me, s2, s3, s23 = subs[c]
            for rdma in rdmas[c]:
                rdma.wait_recv()
            half = CHUNK // 2
            rdmas[c] = start_rdma(
                work_ref.at[pl.ds(base + keep[c], half), :],
                work_ref.at[pl.ds(base + keep[c], half), :],
                c * SLOTS + 7,
                my ^ masks[0],
            )
            for j, s in enumerate((s2, s3, s23)):
                emit_block(c, 1 + j, keep[c] + s * SUB, SUB)

        for c in range(N_C):
            base, masks = CHUNKS[c]
            rdmas[c].wait_recv()
            b1 = bit_of[masks[0]]
            p_keep = (1 - b1) * (CHUNK // 2)
            emit_block(c, 4, p_keep, CHUNK // 2)

        for rdma in all_rdmas:
            rdma.wait_send()
        for cp in copy_ops:
            cp.wait()

    scratch_shapes = [
        pltpu.VMEM((m, n), jnp.bfloat16),
        pltpu.VMEM((m, n), jnp.float32),
    ] + [
        pltpu.VMEM((CHUNK // 2, n), jnp.bfloat16) for _ in range(N_C)
    ] + [
        pltpu.VMEM((SUB, n), jnp.bfloat16) for _ in range(3 * N_C)
    ] + [
        pltpu.SemaphoreType.DMA((SLOTS * N_C,)),
        pltpu.SemaphoreType.DMA((SLOTS * N_C,)),
        pltpu.SemaphoreType.DMA((5 * N_C,)),
    ]

    return pl.pallas_call(
        body,
        out_shape=jax.ShapeDtypeStruct((m, n), jnp.float32),
        in_specs=[
            pl.BlockSpec(memory_space=pltpu.VMEM),
            pl.BlockSpec(memory_space=pltpu.VMEM),
        ],
        out_specs=pl.BlockSpec(memory_space=pl.ANY),
        scratch_shapes=scratch_shapes,
        compiler_params=pltpu.CompilerParams(
            vmem_limit_bytes=100 * 1024 * 1024,
            collective_id=0,
        ),
    )(A, B)
